# baseline (speedup 1.0000x reference)
import functools
import math

import jax
import jax.numpy as jnp
from jax import lax
from jax.experimental import pallas as pl
from jax.experimental.pallas import tpu as pltpu
from jax.experimental.pallas import tpu_sc as plsc

_LANES = 16


@functools.lru_cache(maxsize=None)
def _build_gather(B, V, D, chunk):
    info = plsc.get_sparse_core_info()
    nc, ns = info.num_cores, info.num_subcores
    nw = nc * ns
    assert B % (nw * chunk) == 0
    b_per_w = B // nw
    n_chunks = b_per_w // chunk
    slices_per_row = D // _LANES
    scale = math.sqrt(D)

    mesh = plsc.VectorSubcoreMesh(core_axis_name="c", subcore_axis_name="s")

    @functools.partial(
        pl.kernel,
        out_type=jax.ShapeDtypeStruct((B, D), jnp.float32),
        mesh=mesh,
        scratch_types=[
            pltpu.VMEM((chunk,), jnp.int32),
            pltpu.VMEM((chunk, D), jnp.float32),
            pltpu.SemaphoreType.DMA,
        ],
        compiler_params=pltpu.CompilerParams(use_tc_tiling_on_sc=False),
    )
    def gather_kernel(idx_hbm, table_hbm, out_hbm, idx_v, rows_v, sem):
        wid = lax.axis_index("s") * nc + lax.axis_index("c")
        base = wid * b_per_w

        @pl.loop(0, n_chunks)
        def _chunk(i):
            cb = base + i * chunk
            pltpu.sync_copy(idx_hbm.at[pl.ds(cb, chunk)], idx_v)
            pltpu.async_copy(table_hbm.at[idx_v], rows_v, sem).wait()

            @pl.loop(0, chunk)
            def _row(r):
                for j in range(slices_per_row):
                    sl = pl.ds(j * _LANES, _LANES)
                    rows_v[r, sl] = rows_v[r, sl] * scale

            pltpu.sync_copy(rows_v, out_hbm.at[pl.ds(cb, chunk)])

    return gather_kernel


def kernel(x, table):
    B = x.shape[0] * x.shape[1]
    V, D = table.shape
    xf = x.reshape(B).astype(jnp.int32)
    out = _build_gather(B, V, D, 1024)(xf, table)
    return out.reshape(x.shape + (D,))


# speedup vs baseline: 1.1050x; 1.1050x over previous
import functools
import math

import jax
import jax.numpy as jnp
from jax import lax
from jax.experimental import pallas as pl
from jax.experimental.pallas import tpu as pltpu
from jax.experimental.pallas import tpu_sc as plsc

_LANES = 16
_NBUF = 2


@functools.lru_cache(maxsize=None)
def _build_gather(B, V, D, chunk):
    info = plsc.get_sparse_core_info()
    nc, ns = info.num_cores, info.num_subcores
    nw = nc * ns
    assert B % (nw * chunk) == 0
    b_per_w = B // nw
    n_chunks = b_per_w // chunk
    assert n_chunks % _NBUF == 0 and n_chunks >= 2 * _NBUF
    slices_per_row = D // _LANES
    scale = math.sqrt(D)

    mesh = plsc.VectorSubcoreMesh(core_axis_name="c", subcore_axis_name="s")

    @functools.partial(
        pl.kernel,
        out_type=jax.ShapeDtypeStruct((B, D), jnp.float32),
        mesh=mesh,
        scratch_types=[
            [pltpu.VMEM((chunk,), jnp.int32) for _ in range(_NBUF)],
            [pltpu.VMEM((chunk, D), jnp.float32) for _ in range(_NBUF)],
            [pltpu.SemaphoreType.DMA for _ in range(_NBUF)],
            [pltpu.SemaphoreType.DMA for _ in range(_NBUF)],
        ],
        compiler_params=pltpu.CompilerParams(use_tc_tiling_on_sc=False),
    )
    def gather_kernel(idx_hbm, table_hbm, out_hbm, idx_v, rows_v, gsem, ssem):
        wid = lax.axis_index("s") * nc + lax.axis_index("c")
        base = wid * b_per_w

        for b in range(_NBUF):
            pltpu.sync_copy(idx_hbm.at[pl.ds(base + b * chunk, chunk)], idx_v[b])
            pltpu.async_copy(table_hbm.at[idx_v[b]], rows_v[b], gsem[b])

        @pl.loop(0, n_chunks, step=_NBUF)
        def _group(g):
            for b in range(_NBUF):
                i = g + b
                pltpu.make_async_copy(table_hbm.at[idx_v[b]], rows_v[b], gsem[b]).wait()

                @plsc.parallel_loop(0, chunk, unroll=2)
                def _row(r):
                    for j in range(slices_per_row):
                        sl = pl.ds(j * _LANES, _LANES)
                        rows_v[b][r, sl] = rows_v[b][r, sl] * scale

                out_slice = out_hbm.at[pl.ds(base + i * chunk, chunk)]
                pltpu.async_copy(rows_v[b], out_slice, ssem[b])

                nxt = i + _NBUF

                @pl.when(nxt < n_chunks)
                def _prefetch():
                    pltpu.sync_copy(
                        idx_hbm.at[pl.ds(base + nxt * chunk, chunk)], idx_v[b]
                    )
                    pltpu.make_async_copy(rows_v[b], out_slice, ssem[b]).wait()
                    pltpu.async_copy(table_hbm.at[idx_v[b]], rows_v[b], gsem[b])

                @pl.when(nxt >= n_chunks)
                def _drain():
                    pltpu.make_async_copy(rows_v[b], out_slice, ssem[b]).wait()

    return gather_kernel


def kernel(x, table):
    B = x.shape[0] * x.shape[1]
    V, D = table.shape
    xf = x.reshape(B).astype(jnp.int32)
    out = _build_gather(B, V, D, 800)(xf, table)
    return out.reshape(x.shape + (D,))
